# baseline (speedup 1.0000x reference)
import jax
import jax.numpy as jnp
from jax import lax
from jax.experimental import pallas as pl
from jax.experimental.pallas import tpu as pltpu
from jax.experimental.pallas import tpu_sc as plsc

VOCAB = 1000000
DIM = 64
BATCH = 4096
SEQ = 50

NC = 2
NS = 16
L = 16
NW = NC * NS
ROWS_PER_W = BATCH // NW
CB = 16
TOK_CB = CB * SEQ
N_CHUNKS = ROWS_PER_W // CB

TCOL = 128
NFULL = VOCAB // TCOL
REM = VOCAB - NFULL * TCOL
NIDX = NFULL // NW + 1
NPAIR = (NIDX + 1) // 2


def _widx():
    return lax.axis_index("s") * NC + lax.axis_index("c")


def _transpose_blk(blk, outv, width):
    iota = lax.iota(jnp.int32, L)
    idx_d = [iota + 16 * g for g in range(4)]
    for r in range(width):
        idx_r = jnp.full((L,), r, jnp.int32)
        for g in range(4):
            v = plsc.load_gather(blk, [idx_d[g], idx_r])
            outv[pl.ds(r * DIM + 16 * g, L)] = v


def _tr_body(embT_hbm, out_hbm,
             blk0, blk1, outv0, outv1, blkr, outr,
             sem_i0, sem_i1, sem_o0, sem_o1):
    wid = _widx()

    def fetch(col, blk, sem):
        pltpu.async_copy(embT_hbm.at[:, pl.ds(col * TCOL, TCOL)], blk, sem)

    def wait_fetch(col, blk, sem):
        pltpu.make_async_copy(embT_hbm.at[:, pl.ds(col * TCOL, TCOL)],
                              blk, sem).wait()

    def put(idx, outv, sem):
        pltpu.async_copy(
            outv, out_hbm.at[pl.ds(idx * (TCOL * DIM), TCOL * DIM)], sem)

    def wait_put(idx, outv, sem):
        pltpu.make_async_copy(
            outv, out_hbm.at[pl.ds(idx * (TCOL * DIM), TCOL * DIM)],
            sem).wait()

    fetch(wid, blk0, sem_i0)

    @pl.loop(0, NPAIR)
    def _pair(i):
        ca = wid + (2 * i) * NW
        cb = wid + (2 * i + 1) * NW
        cn = wid + (2 * i + 2) * NW

        @pl.when(cb < NFULL)
        def _():
            fetch(cb, blk1, sem_i1)

        @pl.when(ca < NFULL)
        def _():
            wait_fetch(ca, blk0, sem_i0)
            _transpose_blk(blk0, outv0, TCOL)
            put(ca, outv0, sem_o0)

        @pl.when(cn < NFULL)
        def _():
            fetch(cn, blk0, sem_i0)

        @pl.when(cb < NFULL)
        def _():
            wait_fetch(cb, blk1, sem_i1)
            _transpose_blk(blk1, outv1, TCOL)
            put(cb, outv1, sem_o1)

        @pl.when(ca < NFULL)
        def _():
            wait_put(ca, outv0, sem_o0)

        @pl.when(cb < NFULL)
        def _():
            wait_put(cb, outv1, sem_o1)

    @pl.when(wid == 0)
    def _rem():
        pltpu.async_copy(embT_hbm.at[:, pl.ds(NFULL * TCOL, REM)],
                         blkr, sem_i0).wait()
        _transpose_blk(blkr, outr, REM)
        pltpu.async_copy(
            outr, out_hbm.at[pl.ds(NFULL * TCOL * DIM, REM * DIM)],
            sem_o0).wait()


def _mesh():
    return plsc.VectorSubcoreMesh(core_axis_name="c", subcore_axis_name="s",
                                  num_cores=NC, num_subcores=NS)


def _transpose_table(embT):
    return pl.kernel(
        _tr_body,
        out_type=jax.ShapeDtypeStruct((VOCAB * DIM,), jnp.float32),
        mesh=_mesh(),
        compiler_params=pltpu.CompilerParams(needs_layout_passes=False,
                                             use_tc_tiling_on_sc=True),
        scratch_types=[
            pltpu.VMEM((DIM, TCOL), jnp.float32),
            pltpu.VMEM((DIM, TCOL), jnp.float32),
            pltpu.VMEM((TCOL * DIM,), jnp.float32),
            pltpu.VMEM((TCOL * DIM,), jnp.float32),
            pltpu.VMEM((DIM, REM), jnp.float32),
            pltpu.VMEM((REM * DIM,), jnp.float32),
            pltpu.SemaphoreType.DMA,
            pltpu.SemaphoreType.DMA,
            pltpu.SemaphoreType.DMA,
            pltpu.SemaphoreType.DMA,
        ],
    )(embT)


def _rsqrt(x):
    i = lax.bitcast_convert_type(x, jnp.int32)
    y = lax.bitcast_convert_type(jnp.int32(0x5F3759DF) - (i >> 1), jnp.float32)
    for _ in range(3):
        y = y * (1.5 - 0.5 * x * y * y)
    return y


def _enc_body(tok_hbm, msk_hbm, embed_hbm, lnw_hbm, lnb_hbm, out_hbm,
              idx_v, msk_v, rows_v, out_v, lnw_v, lnb_v, sem):
    wid = _widx()
    pltpu.sync_copy(lnw_hbm, lnw_v)
    pltpu.sync_copy(lnb_hbm, lnb_v)

    @pl.loop(0, N_CHUNKS)
    def _chunk(c):
        row0 = pl.multiple_of(wid * ROWS_PER_W + c * CB, 16)

        pltpu.sync_copy(tok_hbm.at[pl.ds(row0, CB)], idx_v)
        pltpu.sync_copy(msk_hbm.at[pl.ds(row0, CB)], msk_v)
        copies = [
            pltpu.async_copy(embed_hbm.at[idx_v.at[b]],
                             rows_v.at[pl.ds(b * SEQ, SEQ)], sem)
            for b in range(CB)
        ]
        for cp in copies:
            cp.wait()

        for b in range(CB):
            base = b * SEQ
            bvec = jnp.full((L,), b, jnp.int32)

            def _tok(s, carry):
                a0, a1, a2, a3, cnt = carry
                w = plsc.load_gather(
                    msk_v, [bvec, jnp.full((L,), s, jnp.int32)]
                ).astype(jnp.float32)
                a0 = a0 + w * rows_v[base + s, pl.ds(0, L)]
                a1 = a1 + w * rows_v[base + s, pl.ds(L, L)]
                a2 = a2 + w * rows_v[base + s, pl.ds(2 * L, L)]
                a3 = a3 + w * rows_v[base + s, pl.ds(3 * L, L)]
                return a0, a1, a2, a3, cnt + w

            z = jnp.zeros((L,), jnp.float32)
            a0, a1, a2, a3, cnt = lax.fori_loop(
                0, SEQ, _tok, (z, z, z, z, z), unroll=5)

            inv = 1.0 / jnp.maximum(cnt, 1.0)
            p0, p1, p2, p3 = a0 * inv, a1 * inv, a2 * inv, a3 * inv
            m = jnp.sum(p0 + p1 + p2 + p3) * (1.0 / DIM)
            d0, d1, d2, d3 = p0 - m, p1 - m, p2 - m, p3 - m
            var = jnp.sum(d0 * d0 + d1 * d1 + d2 * d2 + d3 * d3) * (1.0 / DIM)
            r = _rsqrt(jnp.full((L,), 1.0, jnp.float32) * (var + 1e-5))
            out_v[b, pl.ds(0, L)] = d0 * r * lnw_v[pl.ds(0, L)] + lnb_v[pl.ds(0, L)]
            out_v[b, pl.ds(L, L)] = d1 * r * lnw_v[pl.ds(L, L)] + lnb_v[pl.ds(L, L)]
            out_v[b, pl.ds(2 * L, L)] = d2 * r * lnw_v[pl.ds(2 * L, L)] + lnb_v[pl.ds(2 * L, L)]
            out_v[b, pl.ds(3 * L, L)] = d3 * r * lnw_v[pl.ds(3 * L, L)] + lnb_v[pl.ds(3 * L, L)]

        pltpu.sync_copy(out_v, out_hbm.at[pl.ds(row0, CB)])


def _encoder(tok, msk, table, ln_weight, ln_bias):
    return pl.kernel(
        _enc_body,
        out_type=jax.ShapeDtypeStruct((BATCH, DIM), jnp.float32),
        mesh=_mesh(),
        compiler_params=pltpu.CompilerParams(needs_layout_passes=False,
                                             use_tc_tiling_on_sc=False),
        scratch_types=[
            pltpu.VMEM((CB, SEQ), jnp.int32),
            pltpu.VMEM((CB, SEQ), jnp.int32),
            pltpu.VMEM((TOK_CB, DIM), jnp.float32),
            pltpu.VMEM((CB, DIM), jnp.float32),
            pltpu.VMEM((DIM,), jnp.float32),
            pltpu.VMEM((DIM,), jnp.float32),
            pltpu.SemaphoreType.DMA,
        ],
    )(tok, msk, table, ln_weight, ln_bias)


@jax.jit
def _run(token_ids, attention_mask, embed, ln_weight, ln_bias):
    flat = _transpose_table(embed.T)
    table = flat.reshape(VOCAB, DIM)
    return _encoder(token_ids.astype(jnp.int32),
                    attention_mask.astype(jnp.int32),
                    table, ln_weight, ln_bias)


def kernel(token_ids, attention_mask, embed, ln_weight, ln_bias):
    return _run(token_ids, attention_mask, embed, ln_weight, ln_bias)


# speedup vs baseline: 1.9085x; 1.9085x over previous
import jax
import jax.numpy as jnp
from jax import lax
from jax.experimental import pallas as pl
from jax.experimental.pallas import tpu as pltpu
from jax.experimental.pallas import tpu_sc as plsc

VOCAB = 1000000
DIM = 64
BATCH = 4096
SEQ = 50

NC = 2
NS = 16
L = 16
NW = NC * NS
ROWS_PER_W = BATCH // NW
CB = 16
TOK_CB = CB * SEQ
N_CHUNKS = ROWS_PER_W // CB

TCOL = 128
NFULL = VOCAB // TCOL
REM = VOCAB - NFULL * TCOL
NIDX = NFULL // NW + 1
NPAIR = (NIDX + 1) // 2


def _widx():
    return lax.axis_index("s") * NC + lax.axis_index("c")


def _transpose_blk(blk, outv, width):
    iota = lax.iota(jnp.int32, L)
    idx_d = [iota + 16 * g for g in range(4)]
    for r0 in range(width):
        idx_r = (iota + r0) & (width - 1)
        t = idx_r * DIM
        for g in range(4):
            v = plsc.load_gather(blk, [idx_d[g], idx_r])
            plsc.store_scatter(outv, [t + idx_d[g]], v)


def _tr_body(embT_hbm, out_hbm,
             blk0, blk1, outv0, outv1, blkr, outr,
             sem_i0, sem_i1, sem_o0, sem_o1):
    wid = _widx()

    def fetch(col, blk, sem):
        pltpu.async_copy(embT_hbm.at[:, pl.ds(col * TCOL, TCOL)], blk, sem)

    def wait_fetch(col, blk, sem):
        pltpu.make_async_copy(embT_hbm.at[:, pl.ds(col * TCOL, TCOL)],
                              blk, sem).wait()

    def put(idx, outv, sem):
        pltpu.async_copy(
            outv, out_hbm.at[pl.ds(idx * (TCOL * DIM), TCOL * DIM)], sem)

    def wait_put(idx, outv, sem):
        pltpu.make_async_copy(
            outv, out_hbm.at[pl.ds(idx * (TCOL * DIM), TCOL * DIM)],
            sem).wait()

    fetch(wid, blk0, sem_i0)

    @pl.loop(0, NPAIR)
    def _pair(i):
        ca = wid + (2 * i) * NW
        cb = wid + (2 * i + 1) * NW
        cn = wid + (2 * i + 2) * NW

        @pl.when(cb < NFULL)
        def _():
            fetch(cb, blk1, sem_i1)

        @pl.when(ca < NFULL)
        def _():
            wait_fetch(ca, blk0, sem_i0)
            _transpose_blk(blk0, outv0, TCOL)
            put(ca, outv0, sem_o0)

        @pl.when(cn < NFULL)
        def _():
            fetch(cn, blk0, sem_i0)

        @pl.when(cb < NFULL)
        def _():
            wait_fetch(cb, blk1, sem_i1)
            _transpose_blk(blk1, outv1, TCOL)
            put(cb, outv1, sem_o1)

        @pl.when(ca < NFULL)
        def _():
            wait_put(ca, outv0, sem_o0)

        @pl.when(cb < NFULL)
        def _():
            wait_put(cb, outv1, sem_o1)

    @pl.when(wid == 0)
    def _rem():
        pltpu.async_copy(embT_hbm.at[:, pl.ds(NFULL * TCOL, REM)],
                         blkr, sem_i0).wait()
        _transpose_blk(blkr, outr, REM)
        pltpu.async_copy(
            outr, out_hbm.at[pl.ds(NFULL * TCOL * DIM, REM * DIM)],
            sem_o0).wait()


def _mesh():
    return plsc.VectorSubcoreMesh(core_axis_name="c", subcore_axis_name="s",
                                  num_cores=NC, num_subcores=NS)


def _transpose_table(embT):
    return pl.kernel(
        _tr_body,
        out_type=jax.ShapeDtypeStruct((VOCAB * DIM,), jnp.float32),
        mesh=_mesh(),
        compiler_params=pltpu.CompilerParams(needs_layout_passes=False,
                                             use_tc_tiling_on_sc=True),
        scratch_types=[
            pltpu.VMEM((DIM, TCOL), jnp.float32),
            pltpu.VMEM((DIM, TCOL), jnp.float32),
            pltpu.VMEM((TCOL * DIM,), jnp.float32),
            pltpu.VMEM((TCOL * DIM,), jnp.float32),
            pltpu.VMEM((DIM, REM), jnp.float32),
            pltpu.VMEM((REM * DIM,), jnp.float32),
            pltpu.SemaphoreType.DMA,
            pltpu.SemaphoreType.DMA,
            pltpu.SemaphoreType.DMA,
            pltpu.SemaphoreType.DMA,
        ],
    )(embT)


def _rsqrt(x):
    i = lax.bitcast_convert_type(x, jnp.int32)
    y = lax.bitcast_convert_type(jnp.int32(0x5F3759DF) - (i >> 1), jnp.float32)
    for _ in range(3):
        y = y * (1.5 - 0.5 * x * y * y)
    return y


def _enc_body(tok_hbm, msk_hbm, embed_hbm, lnw_hbm, lnb_hbm, out_hbm,
              idx_v, msk_v, rows_v, out_v, lnw_v, lnb_v, sem):
    wid = _widx()
    pltpu.sync_copy(lnw_hbm, lnw_v)
    pltpu.sync_copy(lnb_hbm, lnb_v)

    @pl.loop(0, N_CHUNKS)
    def _chunk(c):
        row0 = pl.multiple_of(wid * ROWS_PER_W + c * CB, 16)

        pltpu.sync_copy(tok_hbm.at[pl.ds(row0, CB)], idx_v)
        pltpu.sync_copy(msk_hbm.at[pl.ds(row0, CB)], msk_v)
        copies = [
            pltpu.async_copy(embed_hbm.at[idx_v.at[b]],
                             rows_v.at[pl.ds(b * SEQ, SEQ)], sem)
            for b in range(CB)
        ]
        for cp in copies:
            cp.wait()

        for b in range(CB):
            base = b * SEQ
            bvec = jnp.full((L,), b, jnp.int32)

            def _tok(s, carry):
                a0, a1, a2, a3, cnt = carry
                w = plsc.load_gather(
                    msk_v, [bvec, jnp.full((L,), s, jnp.int32)]
                ).astype(jnp.float32)
                a0 = a0 + w * rows_v[base + s, pl.ds(0, L)]
                a1 = a1 + w * rows_v[base + s, pl.ds(L, L)]
                a2 = a2 + w * rows_v[base + s, pl.ds(2 * L, L)]
                a3 = a3 + w * rows_v[base + s, pl.ds(3 * L, L)]
                return a0, a1, a2, a3, cnt + w

            z = jnp.zeros((L,), jnp.float32)
            a0, a1, a2, a3, cnt = lax.fori_loop(
                0, SEQ, _tok, (z, z, z, z, z), unroll=5)

            inv = 1.0 / jnp.maximum(cnt, 1.0)
            p0, p1, p2, p3 = a0 * inv, a1 * inv, a2 * inv, a3 * inv
            m = jnp.sum(p0 + p1 + p2 + p3) * (1.0 / DIM)
            d0, d1, d2, d3 = p0 - m, p1 - m, p2 - m, p3 - m
            var = jnp.sum(d0 * d0 + d1 * d1 + d2 * d2 + d3 * d3) * (1.0 / DIM)
            r = _rsqrt(jnp.full((L,), 1.0, jnp.float32) * (var + 1e-5))
            out_v[b, pl.ds(0, L)] = d0 * r * lnw_v[pl.ds(0, L)] + lnb_v[pl.ds(0, L)]
            out_v[b, pl.ds(L, L)] = d1 * r * lnw_v[pl.ds(L, L)] + lnb_v[pl.ds(L, L)]
            out_v[b, pl.ds(2 * L, L)] = d2 * r * lnw_v[pl.ds(2 * L, L)] + lnb_v[pl.ds(2 * L, L)]
            out_v[b, pl.ds(3 * L, L)] = d3 * r * lnw_v[pl.ds(3 * L, L)] + lnb_v[pl.ds(3 * L, L)]

        pltpu.sync_copy(out_v, out_hbm.at[pl.ds(row0, CB)])


def _encoder(tok, msk, table, ln_weight, ln_bias):
    return pl.kernel(
        _enc_body,
        out_type=jax.ShapeDtypeStruct((BATCH, DIM), jnp.float32),
        mesh=_mesh(),
        compiler_params=pltpu.CompilerParams(needs_layout_passes=False,
                                             use_tc_tiling_on_sc=False),
        scratch_types=[
            pltpu.VMEM((CB, SEQ), jnp.int32),
            pltpu.VMEM((CB, SEQ), jnp.int32),
            pltpu.VMEM((TOK_CB, DIM), jnp.float32),
            pltpu.VMEM((CB, DIM), jnp.float32),
            pltpu.VMEM((DIM,), jnp.float32),
            pltpu.VMEM((DIM,), jnp.float32),
            pltpu.SemaphoreType.DMA,
        ],
    )(tok, msk, table, ln_weight, ln_bias)


@jax.jit
def _run(token_ids, attention_mask, embed, ln_weight, ln_bias):
    flat = _transpose_table(embed.T)
    table = flat.reshape(VOCAB, DIM)
    return _encoder(token_ids.astype(jnp.int32),
                    attention_mask.astype(jnp.int32),
                    table, ln_weight, ln_bias)


def kernel(token_ids, attention_mask, embed, ln_weight, ln_bias):
    return _run(token_ids, attention_mask, embed, ln_weight, ln_bias)


# speedup vs baseline: 4.8948x; 2.5647x over previous
import jax
import jax.numpy as jnp
from jax import lax
from jax.experimental import pallas as pl
from jax.experimental.pallas import tpu as pltpu
from jax.experimental.pallas import tpu_sc as plsc

VOCAB = 1000000
DIM = 64
BATCH = 4096
SEQ = 50

NC = 2
NS = 16
L = 16
NW = NC * NS
ROWS_PER_W = BATCH // NW
CB = 16
TOK_CB = CB * SEQ
N_CHUNKS = ROWS_PER_W // CB

TCOL = 128
NFULL = VOCAB // TCOL
REM = VOCAB - NFULL * TCOL
NIDX = NFULL // NW + 1
NPAIR = (NIDX + 1) // 2


def _widx():
    return lax.axis_index("s") * NC + lax.axis_index("c")


def _transpose_blk(blk, outv, width):
    iota = lax.iota(jnp.int32, L)
    idx_d = [iota + 16 * g for g in range(4)]

    @plsc.parallel_loop(0, width, unroll=8)
    def _row(r0):
        idx_r = (iota + r0) & (width - 1)
        t = idx_r * DIM
        for g in range(4):
            v = plsc.load_gather(blk, [idx_d[g], idx_r])
            plsc.store_scatter(outv, [t + idx_d[g]], v)


def _tr_body(embT_hbm, out_hbm,
             blk0, blk1, outv0, outv1, blkr, outr,
             sem_i0, sem_i1, sem_o0, sem_o1):
    wid = _widx()

    def fetch(col, blk, sem):
        pltpu.async_copy(embT_hbm.at[:, pl.ds(col * TCOL, TCOL)], blk, sem)

    def wait_fetch(col, blk, sem):
        pltpu.make_async_copy(embT_hbm.at[:, pl.ds(col * TCOL, TCOL)],
                              blk, sem).wait()

    def put(idx, outv, sem):
        pltpu.async_copy(
            outv, out_hbm.at[pl.ds(idx * (TCOL * DIM), TCOL * DIM)], sem)

    def wait_put(idx, outv, sem):
        pltpu.make_async_copy(
            outv, out_hbm.at[pl.ds(idx * (TCOL * DIM), TCOL * DIM)],
            sem).wait()

    fetch(wid, blk0, sem_i0)

    @pl.loop(0, NPAIR)
    def _pair(i):
        ca = wid + (2 * i) * NW
        cb = wid + (2 * i + 1) * NW
        cn = wid + (2 * i + 2) * NW

        @pl.when(cb < NFULL)
        def _():
            fetch(cb, blk1, sem_i1)

        @pl.when(ca < NFULL)
        def _():
            wait_fetch(ca, blk0, sem_i0)
            _transpose_blk(blk0, outv0, TCOL)
            put(ca, outv0, sem_o0)

        @pl.when(cn < NFULL)
        def _():
            fetch(cn, blk0, sem_i0)

        @pl.when(cb < NFULL)
        def _():
            wait_fetch(cb, blk1, sem_i1)
            _transpose_blk(blk1, outv1, TCOL)
            put(cb, outv1, sem_o1)

        @pl.when(ca < NFULL)
        def _():
            wait_put(ca, outv0, sem_o0)

        @pl.when(cb < NFULL)
        def _():
            wait_put(cb, outv1, sem_o1)

    @pl.when(wid == 0)
    def _rem():
        pltpu.async_copy(embT_hbm.at[:, pl.ds(NFULL * TCOL, REM)],
                         blkr, sem_i0).wait()
        _transpose_blk(blkr, outr, REM)
        pltpu.async_copy(
            outr, out_hbm.at[pl.ds(NFULL * TCOL * DIM, REM * DIM)],
            sem_o0).wait()


def _mesh():
    return plsc.VectorSubcoreMesh(core_axis_name="c", subcore_axis_name="s",
                                  num_cores=NC, num_subcores=NS)


def _transpose_table(embT):
    return pl.kernel(
        _tr_body,
        out_type=jax.ShapeDtypeStruct((VOCAB * DIM,), jnp.float32),
        mesh=_mesh(),
        compiler_params=pltpu.CompilerParams(needs_layout_passes=False,
                                             use_tc_tiling_on_sc=True),
        scratch_types=[
            pltpu.VMEM((DIM, TCOL), jnp.float32),
            pltpu.VMEM((DIM, TCOL), jnp.float32),
            pltpu.VMEM((TCOL * DIM,), jnp.float32),
            pltpu.VMEM((TCOL * DIM,), jnp.float32),
            pltpu.VMEM((DIM, REM), jnp.float32),
            pltpu.VMEM((REM * DIM,), jnp.float32),
            pltpu.SemaphoreType.DMA,
            pltpu.SemaphoreType.DMA,
            pltpu.SemaphoreType.DMA,
            pltpu.SemaphoreType.DMA,
        ],
    )(embT)


def _rsqrt(x):
    i = lax.bitcast_convert_type(x, jnp.int32)
    y = lax.bitcast_convert_type(jnp.int32(0x5F3759DF) - (i >> 1), jnp.float32)
    for _ in range(3):
        y = y * (1.5 - 0.5 * x * y * y)
    return y


def _enc_body(tok_hbm, msk_hbm, embed_hbm, lnw_hbm, lnb_hbm, out_hbm,
              idx_v, msk_v, rows_v, out_v, lnw_v, lnb_v, sem):
    wid = _widx()
    pltpu.sync_copy(lnw_hbm, lnw_v)
    pltpu.sync_copy(lnb_hbm, lnb_v)

    @pl.loop(0, N_CHUNKS)
    def _chunk(c):
        row0 = pl.multiple_of(wid * ROWS_PER_W + c * CB, 16)

        pltpu.sync_copy(tok_hbm.at[pl.ds(row0, CB)], idx_v)
        pltpu.sync_copy(msk_hbm.at[pl.ds(row0, CB)], msk_v)
        copies = [
            pltpu.async_copy(embed_hbm.at[idx_v.at[b]],
                             rows_v.at[pl.ds(b * SEQ, SEQ)], sem)
            for b in range(CB)
        ]
        for cp in copies:
            cp.wait()

        for b in range(CB):
            base = b * SEQ
            bvec = jnp.full((L,), b, jnp.int32)

            def _tok(s, carry):
                a0, a1, a2, a3, cnt = carry
                w = plsc.load_gather(
                    msk_v, [bvec, jnp.full((L,), s, jnp.int32)]
                ).astype(jnp.float32)
                a0 = a0 + w * rows_v[base + s, pl.ds(0, L)]
                a1 = a1 + w * rows_v[base + s, pl.ds(L, L)]
                a2 = a2 + w * rows_v[base + s, pl.ds(2 * L, L)]
                a3 = a3 + w * rows_v[base + s, pl.ds(3 * L, L)]
                return a0, a1, a2, a3, cnt + w

            z = jnp.zeros((L,), jnp.float32)
            a0, a1, a2, a3, cnt = lax.fori_loop(
                0, SEQ, _tok, (z, z, z, z, z), unroll=5)

            inv = 1.0 / jnp.maximum(cnt, 1.0)
            p0, p1, p2, p3 = a0 * inv, a1 * inv, a2 * inv, a3 * inv
            m = jnp.sum(p0 + p1 + p2 + p3) * (1.0 / DIM)
            d0, d1, d2, d3 = p0 - m, p1 - m, p2 - m, p3 - m
            var = jnp.sum(d0 * d0 + d1 * d1 + d2 * d2 + d3 * d3) * (1.0 / DIM)
            r = _rsqrt(jnp.full((L,), 1.0, jnp.float32) * (var + 1e-5))
            out_v[b, pl.ds(0, L)] = d0 * r * lnw_v[pl.ds(0, L)] + lnb_v[pl.ds(0, L)]
            out_v[b, pl.ds(L, L)] = d1 * r * lnw_v[pl.ds(L, L)] + lnb_v[pl.ds(L, L)]
            out_v[b, pl.ds(2 * L, L)] = d2 * r * lnw_v[pl.ds(2 * L, L)] + lnb_v[pl.ds(2 * L, L)]
            out_v[b, pl.ds(3 * L, L)] = d3 * r * lnw_v[pl.ds(3 * L, L)] + lnb_v[pl.ds(3 * L, L)]

        pltpu.sync_copy(out_v, out_hbm.at[pl.ds(row0, CB)])


def _encoder(tok, msk, table, ln_weight, ln_bias):
    return pl.kernel(
        _enc_body,
        out_type=jax.ShapeDtypeStruct((BATCH, DIM), jnp.float32),
        mesh=_mesh(),
        compiler_params=pltpu.CompilerParams(needs_layout_passes=False,
                                             use_tc_tiling_on_sc=False),
        scratch_types=[
            pltpu.VMEM((CB, SEQ), jnp.int32),
            pltpu.VMEM((CB, SEQ), jnp.int32),
            pltpu.VMEM((TOK_CB, DIM), jnp.float32),
            pltpu.VMEM((CB, DIM), jnp.float32),
            pltpu.VMEM((DIM,), jnp.float32),
            pltpu.VMEM((DIM,), jnp.float32),
            pltpu.SemaphoreType.DMA,
        ],
    )(tok, msk, table, ln_weight, ln_bias)


@jax.jit
def _run(token_ids, attention_mask, embed, ln_weight, ln_bias):
    flat = _transpose_table(embed.T)
    table = flat.reshape(VOCAB, DIM)
    return _encoder(token_ids.astype(jnp.int32),
                    attention_mask.astype(jnp.int32),
                    table, ln_weight, ln_bias)


def kernel(token_ids, attention_mask, embed, ln_weight, ln_bias):
    return _run(token_ids, attention_mask, embed, ln_weight, ln_bias)


# speedup vs baseline: 5.6574x; 1.1558x over previous
import jax
import jax.numpy as jnp
from jax import lax
from jax.experimental import pallas as pl
from jax.experimental.pallas import tpu as pltpu
from jax.experimental.pallas import tpu_sc as plsc

VOCAB = 1000000
DIM = 64
BATCH = 4096
SEQ = 50

NC = 2
NS = 16
L = 16
NW = NC * NS
ROWS_PER_W = BATCH // NW
CB = 16
TOK_CB = CB * SEQ
N_CHUNKS = ROWS_PER_W // CB

TCOL = 256
NFULL = VOCAB // TCOL
REM = VOCAB - NFULL * TCOL
NIDX = NFULL // NW + 1
NPAIR = (NIDX + 1) // 2


def _widx():
    return lax.axis_index("s") * NC + lax.axis_index("c")


def _transpose_blk(blk, outv, width):
    iota = lax.iota(jnp.int32, L)
    idx_d = [iota + 16 * g for g in range(4)]

    @plsc.parallel_loop(0, width, unroll=8)
    def _row(r0):
        idx_r = (iota + r0) & (width - 1)
        t = idx_r * DIM
        for g in range(4):
            v = plsc.load_gather(blk, [idx_d[g], idx_r])
            plsc.store_scatter(outv, [t + idx_d[g]], v)


def _tr_body(embT_hbm, out_hbm,
             blk0, blk1, outv0, outv1, blkr, outr,
             sem_i0, sem_i1, sem_o0, sem_o1):
    wid = _widx()

    def fetch(col, blk, sem):
        pltpu.async_copy(embT_hbm.at[:, pl.ds(col * TCOL, TCOL)], blk, sem)

    def wait_fetch(col, blk, sem):
        pltpu.make_async_copy(embT_hbm.at[:, pl.ds(col * TCOL, TCOL)],
                              blk, sem).wait()

    def put(idx, outv, sem):
        pltpu.async_copy(
            outv, out_hbm.at[pl.ds(idx * (TCOL * DIM), TCOL * DIM)], sem)

    def wait_put(idx, outv, sem):
        pltpu.make_async_copy(
            outv, out_hbm.at[pl.ds(idx * (TCOL * DIM), TCOL * DIM)],
            sem).wait()

    fetch(wid, blk0, sem_i0)

    @pl.loop(0, NPAIR)
    def _pair(i):
        ca = wid + (2 * i) * NW
        cb = wid + (2 * i + 1) * NW
        cn = wid + (2 * i + 2) * NW

        @pl.when(cb < NFULL)
        def _():
            fetch(cb, blk1, sem_i1)

        @pl.when(ca < NFULL)
        def _():
            wait_fetch(ca, blk0, sem_i0)
            _transpose_blk(blk0, outv0, TCOL)
            put(ca, outv0, sem_o0)

        @pl.when(cn < NFULL)
        def _():
            fetch(cn, blk0, sem_i0)

        @pl.when(cb < NFULL)
        def _():
            wait_fetch(cb, blk1, sem_i1)
            _transpose_blk(blk1, outv1, TCOL)
            put(cb, outv1, sem_o1)

        @pl.when(ca < NFULL)
        def _():
            wait_put(ca, outv0, sem_o0)

        @pl.when(cb < NFULL)
        def _():
            wait_put(cb, outv1, sem_o1)

    @pl.when(wid == 0)
    def _rem():
        pltpu.async_copy(embT_hbm.at[:, pl.ds(NFULL * TCOL, REM)],
                         blkr, sem_i0).wait()
        _transpose_blk(blkr, outr, REM)
        pltpu.async_copy(
            outr, out_hbm.at[pl.ds(NFULL * TCOL * DIM, REM * DIM)],
            sem_o0).wait()


def _mesh():
    return plsc.VectorSubcoreMesh(core_axis_name="c", subcore_axis_name="s",
                                  num_cores=NC, num_subcores=NS)


def _transpose_table(embT):
    return pl.kernel(
        _tr_body,
        out_type=jax.ShapeDtypeStruct((VOCAB * DIM,), jnp.float32),
        mesh=_mesh(),
        compiler_params=pltpu.CompilerParams(needs_layout_passes=False,
                                             use_tc_tiling_on_sc=True),
        scratch_types=[
            pltpu.VMEM((DIM, TCOL), jnp.float32),
            pltpu.VMEM((DIM, TCOL), jnp.float32),
            pltpu.VMEM((TCOL * DIM,), jnp.float32),
            pltpu.VMEM((TCOL * DIM,), jnp.float32),
            pltpu.VMEM((DIM, REM), jnp.float32),
            pltpu.VMEM((REM * DIM,), jnp.float32),
            pltpu.SemaphoreType.DMA,
            pltpu.SemaphoreType.DMA,
            pltpu.SemaphoreType.DMA,
            pltpu.SemaphoreType.DMA,
        ],
    )(embT)


def _rsqrt(x):
    i = lax.bitcast_convert_type(x, jnp.int32)
    y = lax.bitcast_convert_type(jnp.int32(0x5F3759DF) - (i >> 1), jnp.float32)
    for _ in range(3):
        y = y * (1.5 - 0.5 * x * y * y)
    return y


def _enc_body(tok_hbm, msk_hbm, embed_hbm, lnw_hbm, lnb_hbm, out_hbm,
              idx_v, msk_v, rows_v, out_v, lnw_v, lnb_v, sem):
    wid = _widx()
    pltpu.sync_copy(lnw_hbm, lnw_v)
    pltpu.sync_copy(lnb_hbm, lnb_v)

    @pl.loop(0, N_CHUNKS)
    def _chunk(c):
        row0 = pl.multiple_of(wid * ROWS_PER_W + c * CB, 16)

        pltpu.sync_copy(tok_hbm.at[pl.ds(row0, CB)], idx_v)
        pltpu.sync_copy(msk_hbm.at[pl.ds(row0, CB)], msk_v)
        copies = [
            pltpu.async_copy(embed_hbm.at[idx_v.at[b]],
                             rows_v.at[pl.ds(b * SEQ, SEQ)], sem)
            for b in range(CB)
        ]
        for cp in copies:
            cp.wait()

        for b in range(CB):
            base = b * SEQ
            bvec = jnp.full((L,), b, jnp.int32)

            def _tok(s, carry):
                a0, a1, a2, a3, cnt = carry
                w = plsc.load_gather(
                    msk_v, [bvec, jnp.full((L,), s, jnp.int32)]
                ).astype(jnp.float32)
                a0 = a0 + w * rows_v[base + s, pl.ds(0, L)]
                a1 = a1 + w * rows_v[base + s, pl.ds(L, L)]
                a2 = a2 + w * rows_v[base + s, pl.ds(2 * L, L)]
                a3 = a3 + w * rows_v[base + s, pl.ds(3 * L, L)]
                return a0, a1, a2, a3, cnt + w

            z = jnp.zeros((L,), jnp.float32)
            a0, a1, a2, a3, cnt = lax.fori_loop(
                0, SEQ, _tok, (z, z, z, z, z), unroll=5)

            inv = 1.0 / jnp.maximum(cnt, 1.0)
            p0, p1, p2, p3 = a0 * inv, a1 * inv, a2 * inv, a3 * inv
            m = jnp.sum(p0 + p1 + p2 + p3) * (1.0 / DIM)
            d0, d1, d2, d3 = p0 - m, p1 - m, p2 - m, p3 - m
            var = jnp.sum(d0 * d0 + d1 * d1 + d2 * d2 + d3 * d3) * (1.0 / DIM)
            r = _rsqrt(jnp.full((L,), 1.0, jnp.float32) * (var + 1e-5))
            out_v[b, pl.ds(0, L)] = d0 * r * lnw_v[pl.ds(0, L)] + lnb_v[pl.ds(0, L)]
            out_v[b, pl.ds(L, L)] = d1 * r * lnw_v[pl.ds(L, L)] + lnb_v[pl.ds(L, L)]
            out_v[b, pl.ds(2 * L, L)] = d2 * r * lnw_v[pl.ds(2 * L, L)] + lnb_v[pl.ds(2 * L, L)]
            out_v[b, pl.ds(3 * L, L)] = d3 * r * lnw_v[pl.ds(3 * L, L)] + lnb_v[pl.ds(3 * L, L)]

        pltpu.sync_copy(out_v, out_hbm.at[pl.ds(row0, CB)])


def _encoder(tok, msk, table, ln_weight, ln_bias):
    return pl.kernel(
        _enc_body,
        out_type=jax.ShapeDtypeStruct((BATCH, DIM), jnp.float32),
        mesh=_mesh(),
        compiler_params=pltpu.CompilerParams(needs_layout_passes=False,
                                             use_tc_tiling_on_sc=False),
        scratch_types=[
            pltpu.VMEM((CB, SEQ), jnp.int32),
            pltpu.VMEM((CB, SEQ), jnp.int32),
            pltpu.VMEM((TOK_CB, DIM), jnp.float32),
            pltpu.VMEM((CB, DIM), jnp.float32),
            pltpu.VMEM((DIM,), jnp.float32),
            pltpu.VMEM((DIM,), jnp.float32),
            pltpu.SemaphoreType.DMA,
        ],
    )(tok, msk, table, ln_weight, ln_bias)


@jax.jit
def _run(token_ids, attention_mask, embed, ln_weight, ln_bias):
    flat = _transpose_table(embed.T)
    table = flat.reshape(VOCAB, DIM)
    return _encoder(token_ids.astype(jnp.int32),
                    attention_mask.astype(jnp.int32),
                    table, ln_weight, ln_bias)


def kernel(token_ids, attention_mask, embed, ln_weight, ln_bias):
    return _run(token_ids, attention_mask, embed, ln_weight, ln_bias)


# speedup vs baseline: 5.8743x; 1.0383x over previous
import jax
import jax.numpy as jnp
from jax import lax
from jax.experimental import pallas as pl
from jax.experimental.pallas import tpu as pltpu
from jax.experimental.pallas import tpu_sc as plsc

VOCAB = 1000000
DIM = 64
BATCH = 4096
SEQ = 50

NC = 2
NS = 16
L = 16
NW = NC * NS
ROWS_PER_W = BATCH // NW
CB = 16
TOK_CB = CB * SEQ
N_CHUNKS = ROWS_PER_W // CB

TCOL = 256
NFULL = VOCAB // TCOL
REM = VOCAB - NFULL * TCOL
NIDX = NFULL // NW + 1
NPAIR = (NIDX + 1) // 2


def _widx():
    return lax.axis_index("s") * NC + lax.axis_index("c")


def _transpose_blk(blk, outv, width):
    iota = lax.iota(jnp.int32, L)
    idx_d = [iota + 16 * g for g in range(4)]

    @plsc.parallel_loop(0, width, unroll=8)
    def _row(r0):
        idx_r = (iota + r0) & (width - 1)
        t = idx_r * DIM
        for g in range(4):
            v = plsc.load_gather(blk, [idx_d[g], idx_r])
            plsc.store_scatter(outv, [t + idx_d[g]], v)


def _tr_body(embT_hbm, out_hbm,
             blk0, blk1, outv0, outv1, blkr, outr,
             sem_i0, sem_i1, sem_o0, sem_o1):
    wid = _widx()

    def fetch(col, blk, sem):
        pltpu.async_copy(embT_hbm.at[:, pl.ds(col * TCOL, TCOL)], blk, sem)

    def wait_fetch(col, blk, sem):
        pltpu.make_async_copy(embT_hbm.at[:, pl.ds(col * TCOL, TCOL)],
                              blk, sem).wait()

    def put(idx, outv, sem):
        pltpu.async_copy(
            outv, out_hbm.at[pl.ds(idx * (TCOL * DIM), TCOL * DIM)], sem)

    def wait_put(idx, outv, sem):
        pltpu.make_async_copy(
            outv, out_hbm.at[pl.ds(idx * (TCOL * DIM), TCOL * DIM)],
            sem).wait()

    fetch(wid, blk0, sem_i0)

    @pl.loop(0, NPAIR)
    def _pair(i):
        ca = wid + (2 * i) * NW
        cb = wid + (2 * i + 1) * NW
        cn = wid + (2 * i + 2) * NW

        @pl.when(cb < NFULL)
        def _():
            fetch(cb, blk1, sem_i1)

        @pl.when(ca < NFULL)
        def _():
            wait_fetch(ca, blk0, sem_i0)
            _transpose_blk(blk0, outv0, TCOL)
            put(ca, outv0, sem_o0)

        @pl.when(cn < NFULL)
        def _():
            fetch(cn, blk0, sem_i0)

        @pl.when(cb < NFULL)
        def _():
            wait_fetch(cb, blk1, sem_i1)
            _transpose_blk(blk1, outv1, TCOL)
            put(cb, outv1, sem_o1)

        @pl.when(ca < NFULL)
        def _():
            wait_put(ca, outv0, sem_o0)

        @pl.when(cb < NFULL)
        def _():
            wait_put(cb, outv1, sem_o1)

    @pl.when(wid == 0)
    def _rem():
        pltpu.async_copy(embT_hbm.at[:, pl.ds(NFULL * TCOL, REM)],
                         blkr, sem_i0).wait()
        _transpose_blk(blkr, outr, REM)
        pltpu.async_copy(
            outr, out_hbm.at[pl.ds(NFULL * TCOL * DIM, REM * DIM)],
            sem_o0).wait()


def _mesh():
    return plsc.VectorSubcoreMesh(core_axis_name="c", subcore_axis_name="s",
                                  num_cores=NC, num_subcores=NS)


def _transpose_table(embT):
    return pl.kernel(
        _tr_body,
        out_type=jax.ShapeDtypeStruct((VOCAB * DIM,), jnp.float32),
        mesh=_mesh(),
        compiler_params=pltpu.CompilerParams(needs_layout_passes=False,
                                             use_tc_tiling_on_sc=True),
        scratch_types=[
            pltpu.VMEM((DIM, TCOL), jnp.float32),
            pltpu.VMEM((DIM, TCOL), jnp.float32),
            pltpu.VMEM((TCOL * DIM,), jnp.float32),
            pltpu.VMEM((TCOL * DIM,), jnp.float32),
            pltpu.VMEM((DIM, REM), jnp.float32),
            pltpu.VMEM((REM * DIM,), jnp.float32),
            pltpu.SemaphoreType.DMA,
            pltpu.SemaphoreType.DMA,
            pltpu.SemaphoreType.DMA,
            pltpu.SemaphoreType.DMA,
        ],
    )(embT)


def _rsqrt(x):
    i = lax.bitcast_convert_type(x, jnp.int32)
    y = lax.bitcast_convert_type(jnp.int32(0x5F3759DF) - (i >> 1), jnp.float32)
    for _ in range(3):
        y = y * (1.5 - 0.5 * x * y * y)
    return y


def _enc_body(tok_hbm, msk_hbm, embed_hbm, lnw_hbm, lnb_hbm, out_hbm,
              idx0, idx1, msk0, msk1, rows0, rows1,
              out_v, lnw_v, lnb_v, semA, semB):
    wid = _widx()
    pltpu.sync_copy(lnw_hbm, lnw_v)
    pltpu.sync_copy(lnb_hbm, lnb_v)

    def stage(row0, idx_v, msk_v):
        pltpu.sync_copy(tok_hbm.at[pl.ds(row0, CB)], idx_v)
        pltpu.sync_copy(msk_hbm.at[pl.ds(row0, CB)], msk_v)

    def fire(idx_v, rows_v, sem):
        for b in range(CB):
            pltpu.async_copy(embed_hbm.at[idx_v.at[b]],
                             rows_v.at[pl.ds(b * SEQ, SEQ)], sem)

    def drain(idx_v, rows_v, sem):
        for b in range(CB):
            pltpu.make_async_copy(embed_hbm.at[idx_v.at[b]],
                                  rows_v.at[pl.ds(b * SEQ, SEQ)], sem).wait()

    def compute(row0, msk_v, rows_v):
        for b in range(CB):
            base = b * SEQ
            bvec = jnp.full((L,), b, jnp.int32)

            def _tok(s, carry):
                a0, a1, a2, a3, cnt = carry
                w = plsc.load_gather(
                    msk_v, [bvec, jnp.full((L,), s, jnp.int32)]
                ).astype(jnp.float32)
                a0 = a0 + w * rows_v[base + s, pl.ds(0, L)]
                a1 = a1 + w * rows_v[base + s, pl.ds(L, L)]
                a2 = a2 + w * rows_v[base + s, pl.ds(2 * L, L)]
                a3 = a3 + w * rows_v[base + s, pl.ds(3 * L, L)]
                return a0, a1, a2, a3, cnt + w

            z = jnp.zeros((L,), jnp.float32)
            a0, a1, a2, a3, cnt = lax.fori_loop(
                0, SEQ, _tok, (z, z, z, z, z), unroll=5)

            inv = 1.0 / jnp.maximum(cnt, 1.0)
            p0, p1, p2, p3 = a0 * inv, a1 * inv, a2 * inv, a3 * inv
            m = jnp.sum(p0 + p1 + p2 + p3) * (1.0 / DIM)
            d0, d1, d2, d3 = p0 - m, p1 - m, p2 - m, p3 - m
            var = jnp.sum(d0 * d0 + d1 * d1 + d2 * d2 + d3 * d3) * (1.0 / DIM)
            r = _rsqrt(jnp.full((L,), 1.0, jnp.float32) * (var + 1e-5))
            out_v[b, pl.ds(0, L)] = d0 * r * lnw_v[pl.ds(0, L)] + lnb_v[pl.ds(0, L)]
            out_v[b, pl.ds(L, L)] = d1 * r * lnw_v[pl.ds(L, L)] + lnb_v[pl.ds(L, L)]
            out_v[b, pl.ds(2 * L, L)] = d2 * r * lnw_v[pl.ds(2 * L, L)] + lnb_v[pl.ds(2 * L, L)]
            out_v[b, pl.ds(3 * L, L)] = d3 * r * lnw_v[pl.ds(3 * L, L)] + lnb_v[pl.ds(3 * L, L)]

        pltpu.sync_copy(out_v, out_hbm.at[pl.ds(row0, CB)])

    base = wid * ROWS_PER_W
    stage(base, idx0, msk0)
    fire(idx0, rows0, semA)

    @pl.loop(0, N_CHUNKS // 2)
    def _pair(i):
        ra = pl.multiple_of(base + (2 * i) * CB, 16)
        rb = pl.multiple_of(base + (2 * i + 1) * CB, 16)
        rn = pl.multiple_of(base + (2 * i + 2) * CB, 16)

        stage(rb, idx1, msk1)
        fire(idx1, rows1, semB)
        drain(idx0, rows0, semA)
        compute(ra, msk0, rows0)

        @pl.when(2 * i + 2 < N_CHUNKS)
        def _():
            stage(rn, idx0, msk0)
            fire(idx0, rows0, semA)

        drain(idx1, rows1, semB)
        compute(rb, msk1, rows1)


def _encoder(tok, msk, table, ln_weight, ln_bias):
    return pl.kernel(
        _enc_body,
        out_type=jax.ShapeDtypeStruct((BATCH, DIM), jnp.float32),
        mesh=_mesh(),
        compiler_params=pltpu.CompilerParams(needs_layout_passes=False,
                                             use_tc_tiling_on_sc=False),
        scratch_types=[
            pltpu.VMEM((CB, SEQ), jnp.int32),
            pltpu.VMEM((CB, SEQ), jnp.int32),
            pltpu.VMEM((CB, SEQ), jnp.int32),
            pltpu.VMEM((CB, SEQ), jnp.int32),
            pltpu.VMEM((TOK_CB, DIM), jnp.float32),
            pltpu.VMEM((TOK_CB, DIM), jnp.float32),
            pltpu.VMEM((CB, DIM), jnp.float32),
            pltpu.VMEM((DIM,), jnp.float32),
            pltpu.VMEM((DIM,), jnp.float32),
            pltpu.SemaphoreType.DMA,
            pltpu.SemaphoreType.DMA,
        ],
    )(tok, msk, table, ln_weight, ln_bias)


@jax.jit
def _run(token_ids, attention_mask, embed, ln_weight, ln_bias):
    flat = _transpose_table(embed.T)
    table = flat.reshape(VOCAB, DIM)
    return _encoder(token_ids.astype(jnp.int32),
                    attention_mask.astype(jnp.int32),
                    table, ln_weight, ln_bias)


def kernel(token_ids, attention_mask, embed, ln_weight, ln_bias):
    return _run(token_ids, attention_mask, embed, ln_weight, ln_bias)
